# Initial kernel scaffold; baseline (speedup 1.0000x reference)
import jax
import jax.numpy as jnp
from jax.experimental import pallas as pl


def kernel(hidden_states, router_logits, w13_weight, w2_weight):
    raise NotImplementedError("write your pallas kernel here")


# speedup vs baseline: 1.3609x; 1.3609x over previous
import functools

import jax
import jax.numpy as jnp
from jax.experimental import pallas as pl
from jax.experimental.pallas import tpu as pltpu


def _moe_dense_body(logits_ref, x_ref, w1_ref, w3_ref, w2_ref, out_ref,
                    coef_ref, *, num_experts):
    e = pl.program_id(0)
    f = pl.program_id(1)

    @pl.when(f == 0)
    def _():
        logits = logits_ref[...]
        m = jnp.max(logits, axis=1, keepdims=True)
        p = jnp.exp(logits - m)
        probs = p / jnp.sum(p, axis=1, keepdims=True)
        i0 = jnp.argmax(probs, axis=1)
        m0 = jnp.max(probs, axis=1, keepdims=True)
        cols = jax.lax.broadcasted_iota(jnp.int32, probs.shape, 1)
        masked = jnp.where(cols == i0[:, None], -jnp.inf, probs)
        i1 = jnp.argmax(masked, axis=1)
        m1 = jnp.max(masked, axis=1, keepdims=True)
        denom = m0 + m1
        coef_ref[...] = jnp.where(
            i0[:, None] == e, m0 / denom,
            jnp.where(i1[:, None] == e, m1 / denom, 0.0))

    w1 = w1_ref[0].astype(jnp.bfloat16)
    w3 = w3_ref[0].astype(jnp.bfloat16)
    w2 = w2_ref[0].astype(jnp.bfloat16)
    x = x_ref[...]

    dn = (((1,), (1,)), ((), ()))
    gate = jax.lax.dot_general(x, w1, dn, preferred_element_type=jnp.float32)
    up = jax.lax.dot_general(x, w3, dn, preferred_element_type=jnp.float32)
    act = (gate * jax.lax.logistic(gate) * up).astype(jnp.bfloat16)
    y = jax.lax.dot_general(act, w2, dn, preferred_element_type=jnp.float32)
    contrib = coef_ref[...] * y

    first = jnp.logical_and(e == 0, f == 0)

    @pl.when(first)
    def _():
        out_ref[...] = contrib

    @pl.when(jnp.logical_not(first))
    def _():
        out_ref[...] += contrib


def kernel(hidden_states, router_logits, w13_weight, w2_weight):
    T, H = hidden_states.shape
    E = router_logits.shape[1]
    ff = w2_weight.shape[2]
    F = 4
    ffc = ff // F

    xb = hidden_states.astype(jnp.bfloat16)

    out = pl.pallas_call(
        functools.partial(_moe_dense_body, num_experts=E),
        grid=(E, F),
        in_specs=[
            pl.BlockSpec((T, E), lambda e, f: (0, 0)),
            pl.BlockSpec((T, H), lambda e, f: (0, 0)),
            pl.BlockSpec((1, ffc, H), lambda e, f: (e, f, 0)),
            pl.BlockSpec((1, ffc, H), lambda e, f: (e, f + F, 0)),
            pl.BlockSpec((1, H, ffc), lambda e, f: (e, 0, f)),
        ],
        out_specs=pl.BlockSpec((T, H), lambda e, f: (0, 0)),
        out_shape=jax.ShapeDtypeStruct((T, H), jnp.float32),
        scratch_shapes=[pltpu.VMEM((T, 1), jnp.float32)],
        compiler_params=pltpu.CompilerParams(
            dimension_semantics=("arbitrary", "arbitrary"),
            vmem_limit_bytes=100 * 1024 * 1024,
        ),
    )(router_logits, xb, w13_weight, w13_weight, w2_weight)
    return out
